# baseline (speedup 1.0000x reference)
import functools

import jax
import jax.numpy as jnp
from jax import lax
from jax.experimental import pallas as pl
from jax.experimental.pallas import tpu as pltpu
from jax.experimental.pallas import tpu_sc as plsc


def _make_emb_gather(B):
    info = plsc.get_sparse_core_info()
    NC, NS = info.num_cores, info.num_subcores
    NW = NC * NS
    bpw = B // NW
    IC = 128
    nchunks = bpw // IC

    mesh = plsc.VectorSubcoreMesh(core_axis_name="c", subcore_axis_name="s")

    out16 = jax.ShapeDtypeStruct((B, 16), jnp.float32)

    @functools.partial(
        pl.kernel,
        mesh=mesh,
        out_type=(out16, out16, out16),
        compiler_params=pltpu.CompilerParams(use_tc_tiling_on_sc=False),
        scratch_types=[
            pltpu.VMEM((16, IC), jnp.int32),
            pltpu.VMEM((bpw, 16), jnp.float32),
            pltpu.VMEM((bpw, 16), jnp.float32),
            pltpu.VMEM((bpw, 16), jnp.float32),
            pltpu.SemaphoreType.DMA,
        ],
    )
    def emb_gather(cat_hbm, t0_hbm, t1_hbm, t2_hbm,
                   out0_hbm, out1_hbm, out2_hbm,
                   cat_v, r0, r1, r2, sem):
        wid = lax.axis_index("s") * NC + lax.axis_index("c")
        base = wid * bpw
        pltpu.sync_copy(cat_hbm.at[wid], cat_v)
        copies = []
        for k, (tbl, dst) in enumerate(((t0_hbm, r0), (t1_hbm, r1),
                                        (t2_hbm, r2))):
            for j in range(nchunks):
                copies.append(pltpu.async_copy(
                    tbl.at[cat_v.at[k * nchunks + j]],
                    dst.at[pl.ds(j * IC, IC)], sem))
        for c in copies:
            c.wait()
        pltpu.sync_copy(r0, out0_hbm.at[pl.ds(base, bpw)])
        pltpu.sync_copy(r1, out1_hbm.at[pl.ds(base, bpw)])
        pltpu.sync_copy(r2, out2_hbm.at[pl.ds(base, bpw)])

    return emb_gather, NW, nchunks, IC


def _mlp_body(cont_ref, e0_ref, e1_ref, e2_ref, ae_ref, ac_ref, b0_ref,
              w1_ref, b1_ref, w2_ref, b2_ref, w3_ref, b3_ref, w4_ref,
              b4_ref, out_ref):
    emb = jnp.concatenate([e0_ref[...], e1_ref[...], e2_ref[...]], axis=1)
    x0 = (emb @ ae_ref[...] + cont_ref[...] @ ac_ref[...]
          + b0_ref[...])
    h = jnp.maximum(x0, 0.0)
    h = jnp.maximum(h @ w1_ref[...] + b1_ref[...], 0.0)
    h = jnp.maximum(h @ w2_ref[...] + b2_ref[...], 0.0)
    h = jnp.maximum(h @ w3_ref[...] + b3_ref[...], 0.0)
    logits = h @ w4_ref[...] + b4_ref[...]
    out_ref[...] = jax.nn.log_softmax(logits, axis=-1)


def _run_mlp(cont, e0, e1, e2, ae, ac, b0p, w1t, b1p, w2t, b2p, w3t, b3p,
             w4t, b4p, blk):
    B = cont.shape[0]
    grid = (B // blk,)
    row = lambda i: (i, 0)
    whole = lambda i: (0, 0)
    return pl.pallas_call(
        _mlp_body,
        grid=grid,
        in_specs=[
            pl.BlockSpec((blk, cont.shape[1]), row),
            pl.BlockSpec((blk, 16), row),
            pl.BlockSpec((blk, 16), row),
            pl.BlockSpec((blk, 16), row),
            pl.BlockSpec(ae.shape, whole),
            pl.BlockSpec(ac.shape, whole),
            pl.BlockSpec(b0p.shape, whole),
            pl.BlockSpec(w1t.shape, whole),
            pl.BlockSpec(b1p.shape, whole),
            pl.BlockSpec(w2t.shape, whole),
            pl.BlockSpec(b2p.shape, whole),
            pl.BlockSpec(w3t.shape, whole),
            pl.BlockSpec(b3p.shape, whole),
            pl.BlockSpec(w4t.shape, whole),
            pl.BlockSpec(b4p.shape, whole),
        ],
        out_specs=pl.BlockSpec((blk, 2), row),
        out_shape=jax.ShapeDtypeStruct((B, 2), jnp.float32),
    )(cont, e0, e1, e2, ae, ac, b0p, w1t, b1p, w2t, b2p, w3t, b3p, w4t, b4p)


def kernel(cont_data, cat_data, E0, E1, E2, W0, b0, W1, b1, W2, b2, W3, b3,
           W4, b4):
    B = cont_data.shape[0]

    emb_gather, NW, nchunks, IC = _make_emb_gather(B)

    cat32 = cat_data.astype(jnp.int32)
    cat_t = cat32.reshape(NW, nchunks, IC, 3).transpose(0, 3, 1, 2)
    cat_t = cat_t.reshape(NW, 3 * nchunks, IC)
    cat_t = jnp.pad(cat_t, ((0, 0), (0, 16 - 3 * nchunks), (0, 0)))

    t0 = jnp.pad(E0, ((0, 0), (0, 16 - E0.shape[1])))
    t1 = jnp.pad(E1, ((0, 0), (0, 16 - E1.shape[1])))
    t2 = jnp.pad(E2, ((0, 0), (0, 16 - E2.shape[1])))

    e0, e1, e2 = emb_gather(cat_t, t0, t1, t2)

    d0, d1, d2 = E0.shape[1], E1.shape[1], E2.shape[1]
    n0 = W0.shape[0]
    ae = jnp.zeros((48, 256), jnp.float32)
    ae = ae.at[0:d0, 0:n0].set(W0[:, 0:d0].T)
    ae = ae.at[16:16 + d1, 0:n0].set(W0[:, d0:d0 + d1].T)
    ae = ae.at[32:32 + d2, 0:n0].set(W0[:, d0 + d1:d0 + d1 + d2].T)
    nc = cont_data.shape[1]
    ac = jnp.zeros((nc, 256), jnp.float32)
    ac = ac.at[:, 0:n0].set(W0[:, d0 + d1 + d2:].T)
    b0p = jnp.zeros((1, 256), jnp.float32).at[0, :n0].set(b0)

    w1t = jnp.zeros((256, 128), jnp.float32).at[:200, :100].set(W1.T)
    b1p = jnp.zeros((1, 128), jnp.float32).at[0, :100].set(b1)
    w2t = jnp.zeros((128, 128), jnp.float32).at[:100, :50].set(W2.T)
    b2p = jnp.zeros((1, 128), jnp.float32).at[0, :50].set(b2)
    w3t = jnp.zeros((128, 128), jnp.float32).at[:50, :20].set(W3.T)
    b3p = jnp.zeros((1, 128), jnp.float32).at[0, :20].set(b3)
    w4t = jnp.zeros((128, 2), jnp.float32).at[:20, :].set(W4.T)
    b4p = b4.reshape(1, 2)

    return _run_mlp(cont_data, e0, e1, e2, ae, ac, b0p, w1t, b1p, w2t, b2p,
                    w3t, b3p, w4t, b4p, blk=1024)


# speedup vs baseline: 2.2317x; 2.2317x over previous
import functools

import jax
import jax.numpy as jnp
from jax import lax
from jax.experimental import pallas as pl
from jax.experimental.pallas import tpu as pltpu
from jax.experimental.pallas import tpu_sc as plsc

_EW = 20


def _make_emb_gather(B, d0, d1, d2, v0, v1, v2):
    info = plsc.get_sparse_core_info()
    NC, NS = info.num_cores, info.num_subcores
    NW = NC * NS
    bpw = B // NW
    ngrp = bpw // 16

    mesh = plsc.VectorSubcoreMesh(core_axis_name="c", subcore_axis_name="s")

    @functools.partial(
        pl.kernel,
        mesh=mesh,
        out_type=jax.ShapeDtypeStruct((B, _EW), jnp.float32),
        compiler_params=pltpu.CompilerParams(use_tc_tiling_on_sc=False,
                                             needs_layout_passes=False),
        scratch_types=[
            pltpu.VMEM((bpw, 3), jnp.int32),
            pltpu.VMEM((v0, d0), jnp.float32),
            pltpu.VMEM((v1, d1), jnp.float32),
            pltpu.VMEM((v2, d2), jnp.float32),
            pltpu.VMEM((bpw, _EW), jnp.float32),
        ],
    )
    def emb_gather(cat_hbm, t0_hbm, t1_hbm, t2_hbm, out_hbm,
                   cat_v, t0_v, t1_v, t2_v, out_v):
        wid = lax.axis_index("s") * NC + lax.axis_index("c")
        base = wid * bpw
        pltpu.sync_copy(cat_hbm.at[pl.ds(base, bpw)], cat_v)
        pltpu.sync_copy(t0_hbm, t0_v)
        pltpu.sync_copy(t1_hbm, t1_v)
        pltpu.sync_copy(t2_hbm, t2_v)

        lanes = lax.iota(jnp.int32, 16)

        def body(g, carry):
            rows = g * 16 + lanes
            col = lambda c: jnp.full((16,), c, jnp.int32)
            iv0 = plsc.load_gather(cat_v, [rows, col(0)])
            iv1 = plsc.load_gather(cat_v, [rows, col(1)])
            iv2 = plsc.load_gather(cat_v, [rows, col(2)])
            for l in range(d0):
                v = plsc.load_gather(t0_v, [iv0, col(l)])
                plsc.store_scatter(out_v, [rows, col(l)], v)
            for l in range(d1):
                v = plsc.load_gather(t1_v, [iv1, col(l)])
                plsc.store_scatter(out_v, [rows, col(d0 + l)], v)
            for l in range(d2):
                v = plsc.load_gather(t2_v, [iv2, col(l)])
                plsc.store_scatter(out_v, [rows, col(d0 + d1 + l)], v)
            return carry

        lax.fori_loop(0, ngrp, body, 0)
        pltpu.sync_copy(out_v, out_hbm.at[pl.ds(base, bpw)])

    return emb_gather


_NT = (((1,), (1,)), ((), ()))


def _dot(x, w):
    return lax.dot_general(x, w, _NT, preferred_element_type=jnp.float32)


def _mlp_body(cont_ref, xe_ref, w0_ref, b0_ref, w1_ref, b1_ref, w2_ref,
              b2_ref, w3_ref, b3_ref, w4_ref, b4_ref, out_ref):
    w0 = w0_ref[...]
    x0 = (_dot(xe_ref[...], w0[:, 0:_EW]) + _dot(cont_ref[...], w0[:, _EW:])
          + b0_ref[...][None, :])
    h = jnp.maximum(x0, 0.0)
    h = jnp.maximum(_dot(h, w1_ref[...]) + b1_ref[...][None, :], 0.0)
    h = jnp.maximum(_dot(h, w2_ref[...]) + b2_ref[...][None, :], 0.0)
    h = jnp.maximum(_dot(h, w3_ref[...]) + b3_ref[...][None, :], 0.0)
    logits = _dot(h, w4_ref[...]) + b4_ref[...][None, :]
    out_ref[...] = jax.nn.log_softmax(logits, axis=-1)


def _run_mlp(cont, xe, W0, b0, W1, b1, W2, b2, W3, b3, W4, b4, blk):
    B = cont.shape[0]
    grid = (B // blk,)
    row = lambda i: (i, 0)
    whole2 = lambda i: (0, 0)
    whole1 = lambda i: (0,)
    wspec = lambda w: pl.BlockSpec(w.shape, whole2)
    bspec = lambda b: pl.BlockSpec(b.shape, whole1)
    return pl.pallas_call(
        _mlp_body,
        grid=grid,
        in_specs=[
            pl.BlockSpec((blk, cont.shape[1]), row),
            pl.BlockSpec((blk, _EW), row),
            wspec(W0), bspec(b0), wspec(W1), bspec(b1), wspec(W2), bspec(b2),
            wspec(W3), bspec(b3), wspec(W4), bspec(b4),
        ],
        out_specs=pl.BlockSpec((blk, 2), row),
        out_shape=jax.ShapeDtypeStruct((B, 2), jnp.float32),
    )(cont, xe, W0, b0, W1, b1, W2, b2, W3, b3, W4, b4)


def kernel(cont_data, cat_data, E0, E1, E2, W0, b0, W1, b1, W2, b2, W3, b3,
           W4, b4):
    B = cont_data.shape[0]
    emb_gather = _make_emb_gather(B, E0.shape[1], E1.shape[1], E2.shape[1],
                                  E0.shape[0], E1.shape[0], E2.shape[0])
    xe = emb_gather(cat_data.astype(jnp.int32), E0, E1, E2)
    return _run_mlp(cont_data, xe, W0, b0, W1, b1, W2, b2, W3, b3, W4, b4,
                    blk=1024)


# speedup vs baseline: 3.0399x; 1.3621x over previous
import functools

import jax
import jax.numpy as jnp
from jax import lax
from jax.experimental import pallas as pl
from jax.experimental.pallas import tpu as pltpu
from jax.experimental.pallas import tpu_sc as plsc

_EW = 20


def _make_emb_gather(B, d0, d1, d2, v0, v1, v2):
    info = plsc.get_sparse_core_info()
    NC, NS = info.num_cores, info.num_subcores
    NW = NC * NS
    bpw = B // NW
    ngrp = bpw // 16

    mesh = plsc.VectorSubcoreMesh(core_axis_name="c", subcore_axis_name="s")

    @functools.partial(
        pl.kernel,
        mesh=mesh,
        out_type=jax.ShapeDtypeStruct((B, _EW), jnp.float32),
        compiler_params=pltpu.CompilerParams(use_tc_tiling_on_sc=False,
                                             needs_layout_passes=False),
        scratch_types=[
            pltpu.VMEM((3, bpw), jnp.int32),
            pltpu.VMEM((v0, d0), jnp.float32),
            pltpu.VMEM((v1, d1), jnp.float32),
            pltpu.VMEM((v2, d2), jnp.float32),
            pltpu.VMEM((bpw, _EW), jnp.float32),
        ],
    )
    def emb_gather(cat_hbm, t0_hbm, t1_hbm, t2_hbm, out_hbm,
                   cat_v, t0_v, t1_v, t2_v, out_v):
        wid = lax.axis_index("s") * NC + lax.axis_index("c")
        base = wid * bpw
        pltpu.sync_copy(cat_hbm.at[:, pl.ds(base, bpw)], cat_v)
        pltpu.sync_copy(t0_hbm, t0_v)
        pltpu.sync_copy(t1_hbm, t1_v)
        pltpu.sync_copy(t2_hbm, t2_v)

        lanes = lax.iota(jnp.int32, 16)

        def body(g, carry):
            rows = g * 16 + lanes
            col = lambda c: jnp.full((16,), c, jnp.int32)
            iv0 = plsc.load_gather(cat_v, [col(0), rows])
            iv1 = plsc.load_gather(cat_v, [col(1), rows])
            iv2 = plsc.load_gather(cat_v, [col(2), rows])
            for l in range(d0):
                v = plsc.load_gather(t0_v, [iv0, col(l)])
                plsc.store_scatter(out_v, [rows, col(l)], v)
            for l in range(d1):
                v = plsc.load_gather(t1_v, [iv1, col(l)])
                plsc.store_scatter(out_v, [rows, col(d0 + l)], v)
            for l in range(d2):
                v = plsc.load_gather(t2_v, [iv2, col(l)])
                plsc.store_scatter(out_v, [rows, col(d0 + d1 + l)], v)
            return carry

        lax.fori_loop(0, ngrp, body, 0)
        pltpu.sync_copy(out_v, out_hbm.at[pl.ds(base, bpw)])

    return emb_gather


def _dot(x, w):
    return lax.dot_general(x, w, (((1,), (0,)), ((), ())),
                           preferred_element_type=jnp.float32)


def _mlp_body(contt_ref, xe_ref, w0et_ref, w0ct_ref, b0_ref, w1t_ref,
              b1_ref, w2t_ref, b2_ref, w3t_ref, b3_ref, w4t_ref, b4_ref,
              out_ref):
    xc = contt_ref[...].T
    x0 = (_dot(xe_ref[...], w0et_ref[...]) + _dot(xc, w0ct_ref[...])
          + b0_ref[...][None, :])
    h = jnp.maximum(x0, 0.0)
    h = jnp.maximum(_dot(h, w1t_ref[...]) + b1_ref[...][None, :], 0.0)
    h = jnp.maximum(_dot(h, w2t_ref[...]) + b2_ref[...][None, :], 0.0)
    h = jnp.maximum(_dot(h, w3t_ref[...]) + b3_ref[...][None, :], 0.0)
    logits = _dot(h, w4t_ref[...]) + b4_ref[...][None, :]
    out_ref[...] = jax.nn.log_softmax(logits, axis=-1).T


def _run_mlp(contt, xe, w0et, w0ct, b0, w1t, b1, w2t, b2, w3t, b3, w4t, b4,
             blk):
    B = contt.shape[1]
    grid = (B // blk,)
    row = lambda i: (i, 0)
    colb = lambda i: (0, i)
    whole2 = lambda i: (0, 0)
    whole1 = lambda i: (0,)
    wspec = lambda w: pl.BlockSpec(w.shape, whole2)
    bspec = lambda b: pl.BlockSpec(b.shape, whole1)
    outt = pl.pallas_call(
        _mlp_body,
        grid=grid,
        in_specs=[
            pl.BlockSpec((contt.shape[0], blk), colb),
            pl.BlockSpec((blk, _EW), row),
            wspec(w0et), wspec(w0ct), bspec(b0), wspec(w1t), bspec(b1),
            wspec(w2t), bspec(b2), wspec(w3t), bspec(b3), wspec(w4t),
            bspec(b4),
        ],
        out_specs=pl.BlockSpec((2, blk), colb),
        out_shape=jax.ShapeDtypeStruct((2, B), jnp.float32),
    )(contt, xe, w0et, w0ct, b0, w1t, b1, w2t, b2, w3t, b3, w4t, b4)
    return outt.T


def kernel(cont_data, cat_data, E0, E1, E2, W0, b0, W1, b1, W2, b2, W3, b3,
           W4, b4):
    B = cont_data.shape[0]
    emb_gather = _make_emb_gather(B, E0.shape[1], E1.shape[1], E2.shape[1],
                                  E0.shape[0], E1.shape[0], E2.shape[0])
    xe = emb_gather(cat_data.astype(jnp.int32).T, E0, E1, E2)
    w0t = W0.T
    return _run_mlp(cont_data.T, xe, w0t[:_EW], w0t[_EW:], b0, W1.T, b1,
                    W2.T, b2, W3.T, b3, W4.T, b4, blk=1024)


# speedup vs baseline: 3.5116x; 1.1552x over previous
import functools

import jax
import jax.numpy as jnp
from jax import lax
from jax.experimental import pallas as pl
from jax.experimental.pallas import tpu as pltpu
from jax.experimental.pallas import tpu_sc as plsc

_EW = 20


def _make_emb_gather(B, d0, d1, d2, v0, v1, v2):
    info = plsc.get_sparse_core_info()
    NC, NS = info.num_cores, info.num_subcores
    NW = NC * NS
    bpw = B // NW
    ngrp = bpw // 16

    mesh = plsc.VectorSubcoreMesh(core_axis_name="c", subcore_axis_name="s")

    @functools.partial(
        pl.kernel,
        mesh=mesh,
        out_type=jax.ShapeDtypeStruct((B, _EW), jnp.float32),
        compiler_params=pltpu.CompilerParams(use_tc_tiling_on_sc=False,
                                             needs_layout_passes=False),
        scratch_types=[
            pltpu.VMEM((3, bpw), jnp.int32),
            pltpu.VMEM((v0, d0), jnp.float32),
            pltpu.VMEM((v1, d1), jnp.float32),
            pltpu.VMEM((v2, d2), jnp.float32),
            pltpu.VMEM((bpw, _EW), jnp.float32),
            pltpu.SemaphoreType.DMA,
        ],
    )
    def emb_gather(cat_hbm, t0_hbm, t1_hbm, t2_hbm, out_hbm,
                   cat_v, t0_v, t1_v, t2_v, out_v, sem):
        wid = lax.axis_index("s") * NC + lax.axis_index("c")
        base = wid * bpw
        cps = [pltpu.async_copy(cat_hbm.at[:, pl.ds(base, bpw)], cat_v, sem),
               pltpu.async_copy(t0_hbm, t0_v, sem),
               pltpu.async_copy(t1_hbm, t1_v, sem),
               pltpu.async_copy(t2_hbm, t2_v, sem)]
        for c in cps:
            c.wait()

        lanes = lax.iota(jnp.int32, 16)

        def body(g, carry):
            rows = g * 16 + lanes
            col = lambda c: jnp.full((16,), c, jnp.int32)
            iv0 = plsc.load_gather(cat_v, [col(0), rows])
            iv1 = plsc.load_gather(cat_v, [col(1), rows])
            iv2 = plsc.load_gather(cat_v, [col(2), rows])
            for l in range(d0):
                v = plsc.load_gather(t0_v, [iv0, col(l)])
                plsc.store_scatter(out_v, [rows, col(l)], v)
            for l in range(d1):
                v = plsc.load_gather(t1_v, [iv1, col(l)])
                plsc.store_scatter(out_v, [rows, col(d0 + l)], v)
            for l in range(d2):
                v = plsc.load_gather(t2_v, [iv2, col(l)])
                plsc.store_scatter(out_v, [rows, col(d0 + d1 + l)], v)
            return carry

        lax.fori_loop(0, ngrp, body, 0)
        pltpu.sync_copy(out_v, out_hbm.at[pl.ds(base, bpw)])

    return emb_gather


def _dot(x, w):
    return lax.dot_general(x, w, (((1,), (0,)), ((), ())),
                           preferred_element_type=jnp.float32)


def _mlp_body(contt_ref, xe_ref, w0et_ref, w0ct_ref, b0_ref, w1t_ref,
              b1_ref, w2t_ref, b2_ref, w3t_ref, b3_ref, w4t_ref, b4_ref,
              out_ref):
    xc = contt_ref[...].T
    x0 = (_dot(xe_ref[...], w0et_ref[...]) + _dot(xc, w0ct_ref[...])
          + b0_ref[...][None, :])
    h = jnp.maximum(x0, 0.0)
    h = jnp.maximum(_dot(h, w1t_ref[...]) + b1_ref[...][None, :], 0.0)
    h = jnp.maximum(_dot(h, w2t_ref[...]) + b2_ref[...][None, :], 0.0)
    h = jnp.maximum(_dot(h, w3t_ref[...]) + b3_ref[...][None, :], 0.0)
    logits_t = (lax.dot_general(w4t_ref[...], h, (((0,), (1,)), ((), ())),
                                preferred_element_type=jnp.float32)
                + b4_ref[...][:, None])
    out_ref[...] = jax.nn.log_softmax(logits_t, axis=0)


def _run_mlp(contt, xe, w0et, w0ct, b0, w1t, b1, w2t, b2, w3t, b3, w4t, b4,
             blk):
    B = contt.shape[1]
    grid = (B // blk,)
    row = lambda i: (i, 0)
    colb = lambda i: (0, i)
    whole2 = lambda i: (0, 0)
    whole1 = lambda i: (0,)
    wspec = lambda w: pl.BlockSpec(w.shape, whole2)
    bspec = lambda b: pl.BlockSpec(b.shape, whole1)
    outt = pl.pallas_call(
        _mlp_body,
        grid=grid,
        in_specs=[
            pl.BlockSpec((contt.shape[0], blk), colb),
            pl.BlockSpec((blk, _EW), row),
            wspec(w0et), wspec(w0ct), bspec(b0), wspec(w1t), bspec(b1),
            wspec(w2t), bspec(b2), wspec(w3t), bspec(b3), wspec(w4t),
            bspec(b4),
        ],
        out_specs=pl.BlockSpec((2, blk), colb),
        out_shape=jax.ShapeDtypeStruct((2, B), jnp.float32),
    )(contt, xe, w0et, w0ct, b0, w1t, b1, w2t, b2, w3t, b3, w4t, b4)
    return outt.T


def kernel(cont_data, cat_data, E0, E1, E2, W0, b0, W1, b1, W2, b2, W3, b3,
           W4, b4):
    B = cont_data.shape[0]
    emb_gather = _make_emb_gather(B, E0.shape[1], E1.shape[1], E2.shape[1],
                                  E0.shape[0], E1.shape[0], E2.shape[0])
    xe = emb_gather(cat_data.astype(jnp.int32).T, E0, E1, E2)
    w0t = W0.T
    return _run_mlp(cont_data.T, xe, w0t[:_EW], w0t[_EW:], b0, W1.T, b1,
                    W2.T, b2, W3.T, b3, W4.T, b4, blk=2048)


# speedup vs baseline: 3.5791x; 1.0192x over previous
import functools

import jax
import jax.numpy as jnp
from jax import lax
from jax.experimental import pallas as pl
from jax.experimental.pallas import tpu as pltpu
from jax.experimental.pallas import tpu_sc as plsc

_EW = 20


def _make_emb_gather(B, d0, d1, d2, v0, v1, v2):
    info = plsc.get_sparse_core_info()
    NC, NS = info.num_cores, info.num_subcores
    NW = NC * NS
    bpw = B // NW
    ngrp = bpw // 16

    mesh = plsc.VectorSubcoreMesh(core_axis_name="c", subcore_axis_name="s")

    @functools.partial(
        pl.kernel,
        mesh=mesh,
        out_type=jax.ShapeDtypeStruct((B, _EW), jnp.float32),
        compiler_params=pltpu.CompilerParams(use_tc_tiling_on_sc=False,
                                             needs_layout_passes=False),
        scratch_types=[
            pltpu.VMEM((3, bpw), jnp.int32),
            pltpu.VMEM((v0, d0), jnp.float32),
            pltpu.VMEM((v1, d1), jnp.float32),
            pltpu.VMEM((v2, d2), jnp.float32),
            pltpu.VMEM((bpw, _EW), jnp.float32),
            pltpu.SemaphoreType.DMA,
        ],
    )
    def emb_gather(cat_hbm, t0_hbm, t1_hbm, t2_hbm, out_hbm,
                   cat_v, t0_v, t1_v, t2_v, out_v, sem):
        wid = lax.axis_index("s") * NC + lax.axis_index("c")
        base = wid * bpw
        cps = [pltpu.async_copy(cat_hbm.at[:, pl.ds(base, bpw)], cat_v, sem),
               pltpu.async_copy(t0_hbm, t0_v, sem),
               pltpu.async_copy(t1_hbm, t1_v, sem),
               pltpu.async_copy(t2_hbm, t2_v, sem)]
        for c in cps:
            c.wait()

        lanes = lax.iota(jnp.int32, 16)

        def body(g, carry):
            rows = g * 16 + lanes
            col = lambda c: jnp.full((16,), c, jnp.int32)
            iv0 = plsc.load_gather(cat_v, [col(0), rows])
            iv1 = plsc.load_gather(cat_v, [col(1), rows])
            iv2 = plsc.load_gather(cat_v, [col(2), rows])
            for l in range(d0):
                v = plsc.load_gather(t0_v, [iv0, col(l)])
                plsc.store_scatter(out_v, [rows, col(l)], v)
            for l in range(d1):
                v = plsc.load_gather(t1_v, [iv1, col(l)])
                plsc.store_scatter(out_v, [rows, col(d0 + l)], v)
            for l in range(d2):
                v = plsc.load_gather(t2_v, [iv2, col(l)])
                plsc.store_scatter(out_v, [rows, col(d0 + d1 + l)], v)
            return carry

        lax.fori_loop(0, ngrp, body, 0)
        pltpu.sync_copy(out_v, out_hbm.at[pl.ds(base, bpw)])

    return emb_gather


def _dot(x, w):
    return lax.dot_general(x, w, (((1,), (0,)), ((), ())),
                           preferred_element_type=jnp.float32)


def _mlp_body(contt_ref, xe_ref, w0et_ref, w0ct_ref, b0_ref, w1t_ref,
              b1_ref, w2t_ref, b2_ref, w3t_ref, b3_ref, w4t_ref, b4_ref,
              out_ref):
    xc = contt_ref[...].T
    x0 = (_dot(xe_ref[...], w0et_ref[...]) + _dot(xc, w0ct_ref[...])
          + b0_ref[...][None, :])
    h = jnp.maximum(x0, 0.0)
    h = jnp.maximum(_dot(h, w1t_ref[...]) + b1_ref[...][None, :], 0.0)
    h = jnp.maximum(_dot(h, w2t_ref[...]) + b2_ref[...][None, :], 0.0)
    h = jnp.maximum(_dot(h, w3t_ref[...]) + b3_ref[...][None, :], 0.0)
    logits_t = (lax.dot_general(w4t_ref[...], h, (((0,), (1,)), ((), ())),
                                preferred_element_type=jnp.float32)
                + b4_ref[...][:, None])
    out_ref[...] = jax.nn.log_softmax(logits_t, axis=0)


def _run_mlp(contt, xe, w0et, w0ct, b0, w1t, b1, w2t, b2, w3t, b3, w4t, b4,
             blk):
    B = contt.shape[1]
    grid = (B // blk,)
    row = lambda i: (i, 0)
    colb = lambda i: (0, i)
    whole2 = lambda i: (0, 0)
    whole1 = lambda i: (0,)
    wspec = lambda w: pl.BlockSpec(w.shape, whole2)
    bspec = lambda b: pl.BlockSpec(b.shape, whole1)
    outt = pl.pallas_call(
        _mlp_body,
        grid=grid,
        in_specs=[
            pl.BlockSpec((contt.shape[0], blk), colb),
            pl.BlockSpec((blk, _EW), row),
            wspec(w0et), wspec(w0ct), bspec(b0), wspec(w1t), bspec(b1),
            wspec(w2t), bspec(b2), wspec(w3t), bspec(b3), wspec(w4t),
            bspec(b4),
        ],
        out_specs=pl.BlockSpec((2, blk), colb),
        out_shape=jax.ShapeDtypeStruct((2, B), jnp.float32),
    )(contt, xe, w0et, w0ct, b0, w1t, b1, w2t, b2, w3t, b3, w4t, b4)
    return outt.T


def kernel(cont_data, cat_data, E0, E1, E2, W0, b0, W1, b1, W2, b2, W3, b3,
           W4, b4):
    B = cont_data.shape[0]
    emb_gather = _make_emb_gather(B, E0.shape[1], E1.shape[1], E2.shape[1],
                                  E0.shape[0], E1.shape[0], E2.shape[0])
    xe = emb_gather(cat_data.astype(jnp.int32).T, E0, E1, E2)
    w0t = W0.T
    return _run_mlp(cont_data.T, xe, w0t[:_EW], w0t[_EW:], b0, W1.T, b1,
                    W2.T, b2, W3.T, b3, W4.T, b4, blk=4096)


# speedup vs baseline: 3.7799x; 1.0561x over previous
import functools

import jax
import jax.numpy as jnp
from jax import lax
from jax.experimental import pallas as pl
from jax.experimental.pallas import tpu as pltpu
from jax.experimental.pallas import tpu_sc as plsc

_EW = 20


def _make_emb_gather(B, d0, d1, d2, v0, v1, v2):
    info = plsc.get_sparse_core_info()
    NC, NS = info.num_cores, info.num_subcores
    NW = NC * NS
    bpw = B // NW
    ngrp = bpw // 16

    mesh = plsc.VectorSubcoreMesh(core_axis_name="c", subcore_axis_name="s")

    @functools.partial(
        pl.kernel,
        mesh=mesh,
        out_type=jax.ShapeDtypeStruct((B, _EW), jnp.float32),
        compiler_params=pltpu.CompilerParams(use_tc_tiling_on_sc=False,
                                             needs_layout_passes=False),
        scratch_types=[
            pltpu.VMEM((3, bpw), jnp.int32),
            pltpu.VMEM((d0, v0), jnp.float32),
            pltpu.VMEM((d1, v1), jnp.float32),
            pltpu.VMEM((d2, v2), jnp.float32),
            pltpu.VMEM((bpw, _EW), jnp.float32),
            pltpu.SemaphoreType.DMA,
        ],
    )
    def emb_gather(cat_hbm, t0_hbm, t1_hbm, t2_hbm, out_hbm,
                   cat_v, t0_v, t1_v, t2_v, out_v, sem):
        wid = lax.axis_index("s") * NC + lax.axis_index("c")
        base = wid * bpw
        cps = [pltpu.async_copy(cat_hbm.at[:, pl.ds(base, bpw)], cat_v, sem),
               pltpu.async_copy(t0_hbm, t0_v, sem),
               pltpu.async_copy(t1_hbm, t1_v, sem),
               pltpu.async_copy(t2_hbm, t2_v, sem)]
        for c in cps:
            c.wait()

        lanes = lax.iota(jnp.int32, 16)

        def body(g, carry):
            rows = g * 16 + lanes
            col = lambda c: jnp.full((16,), c, jnp.int32)
            iv0 = plsc.load_gather(cat_v, [col(0), rows])
            iv1 = plsc.load_gather(cat_v, [col(1), rows])
            iv2 = plsc.load_gather(cat_v, [col(2), rows])
            for l in range(d0):
                v = plsc.load_gather(t0_v, [col(l), iv0])
                plsc.store_scatter(out_v, [rows, col(l)], v)
            for l in range(d1):
                v = plsc.load_gather(t1_v, [col(l), iv1])
                plsc.store_scatter(out_v, [rows, col(d0 + l)], v)
            for l in range(d2):
                v = plsc.load_gather(t2_v, [col(l), iv2])
                plsc.store_scatter(out_v, [rows, col(d0 + d1 + l)], v)
            return carry

        lax.fori_loop(0, ngrp, body, 0)
        pltpu.sync_copy(out_v, out_hbm.at[pl.ds(base, bpw)])

    return emb_gather


def _dot(x, w):
    return lax.dot_general(x, w, (((1,), (0,)), ((), ())),
                           preferred_element_type=jnp.float32)


def _dotnt(x, w):
    return lax.dot_general(x, w, (((1,), (1,)), ((), ())),
                           preferred_element_type=jnp.float32)


def _mlp_body(contt_ref, xe_ref, w0et_ref, w0ct_ref, b0_ref, w1t_ref,
              b1_ref, w2_ref, b2_ref, w3_ref, b3_ref, w4_ref, b4_ref,
              out_ref):
    xc = contt_ref[...].T
    x0 = (_dot(xe_ref[...], w0et_ref[...]) + _dot(xc, w0ct_ref[...])
          + b0_ref[...][None, :])
    h = jnp.maximum(x0, 0.0)
    h = jnp.maximum(_dot(h, w1t_ref[...]) + b1_ref[...][None, :], 0.0)
    h = jnp.maximum(_dotnt(h, w2_ref[...]) + b2_ref[...][None, :], 0.0)
    h = jnp.maximum(_dotnt(h, w3_ref[...]) + b3_ref[...][None, :], 0.0)
    logits_t = (lax.dot_general(w4_ref[...], h, (((1,), (1,)), ((), ())),
                                preferred_element_type=jnp.float32)
                + b4_ref[...][:, None])
    out_ref[...] = jax.nn.log_softmax(logits_t, axis=0)


def _run_mlp(contt, xe, w0et, w0ct, b0, w1t, b1, w2, b2, w3, b3, w4, b4,
             blk):
    B = contt.shape[1]
    grid = (B // blk,)
    row = lambda i: (i, 0)
    colb = lambda i: (0, i)
    whole2 = lambda i: (0, 0)
    whole1 = lambda i: (0,)
    wspec = lambda w: pl.BlockSpec(w.shape, whole2)
    bspec = lambda b: pl.BlockSpec(b.shape, whole1)
    outt = pl.pallas_call(
        _mlp_body,
        grid=grid,
        in_specs=[
            pl.BlockSpec((contt.shape[0], blk), colb),
            pl.BlockSpec((blk, _EW), row),
            wspec(w0et), wspec(w0ct), bspec(b0), wspec(w1t), bspec(b1),
            wspec(w2), bspec(b2), wspec(w3), bspec(b3), wspec(w4),
            bspec(b4),
        ],
        out_specs=pl.BlockSpec((2, blk), colb),
        out_shape=jax.ShapeDtypeStruct((2, B), jnp.float32),
    )(contt, xe, w0et, w0ct, b0, w1t, b1, w2, b2, w3, b3, w4, b4)
    return outt.T


def kernel(cont_data, cat_data, E0, E1, E2, W0, b0, W1, b1, W2, b2, W3, b3,
           W4, b4):
    B = cont_data.shape[0]
    emb_gather = _make_emb_gather(B, E0.shape[1], E1.shape[1], E2.shape[1],
                                  E0.shape[0], E1.shape[0], E2.shape[0])
    xe = emb_gather(cat_data.astype(jnp.int32).T, E0.T, E1.T, E2.T)
    w0t = W0.T
    return _run_mlp(cont_data.T, xe, w0t[:_EW], w0t[_EW:], b0, W1.T, b1,
                    W2, b2, W3, b3, W4, b4, blk=4096)
